# Initial kernel scaffold; baseline (speedup 1.0000x reference)
import jax
import jax.numpy as jnp
from jax.experimental import pallas as pl


def kernel(input_ids, unigram_table, bigram_table):
    raise NotImplementedError("write your pallas kernel here")


# speedup vs baseline: 1.0279x; 1.0279x over previous
import functools

import jax
import jax.numpy as jnp
from jax import lax
from jax.experimental import pallas as pl
from jax.experimental.pallas import tpu as pltpu
from jax.experimental.pallas import tpu_sc as plsc

HS = 300000
D = 64
NC = 2
NS = 16
L = 16
NW = NC * NS
INV_HS = 1.0 / HS
GW = 128
CHUNK = 512
NG = CHUNK // GW


def _mod_hs(x):
    q = (x.astype(jnp.float32) * INV_HS).astype(jnp.int32)
    r = x - q * HS
    r = jnp.where(r >= HS, r - HS, r)
    r = jnp.where(r < 0, r + HS, r)
    return r


@functools.cache
def _make_kernel(B):
    n_per_w = B // NW
    n_chunks = n_per_w // CHUNK
    mesh = plsc.VectorSubcoreMesh(core_axis_name="c", subcore_axis_name="s")

    @functools.partial(
        pl.kernel,
        out_type=jax.ShapeDtypeStruct((B, D), jnp.float32),
        mesh=mesh,
        compiler_params=pltpu.CompilerParams(use_tc_tiling_on_sc=False),
        scratch_types=[
            pltpu.VMEM((CHUNK,), jnp.int32),
            pltpu.VMEM((CHUNK,), jnp.int32),
            pltpu.VMEM((NG, GW), jnp.int32),
            pltpu.VMEM((NG, GW), jnp.int32),
            pltpu.VMEM((NG, GW), jnp.int32),
            pltpu.VMEM((CHUNK, D), jnp.float32),
            pltpu.VMEM((CHUNK, D), jnp.float32),
            pltpu.VMEM_SHARED((NS * CHUNK, D), jnp.float32),
            pltpu.SemaphoreType.DMA,
            pltpu.SemaphoreType.DMA,
        ],
    )
    def k(ids_hbm, prev_hbm, uni_hbm, bi_hbm, out_hbm,
          ids_v, prev_v, uidx_v, bidx_v, lin_v, uni_v, bi_v, acc_s, sem_u, sem_b):
        wid = lax.axis_index("s") * NC + lax.axis_index("c")
        sid = lax.axis_index("s")
        base_w = wid * n_per_w
        acc_base = sid * CHUNK

        iota16 = lax.iota(jnp.int32, 16)
        for t in range(NG):
            for kk in range(GW // L):
                lin_v[t, pl.ds(kk * L, L)] = iota16 + (acc_base + t * GW + kk * L)

        @pl.loop(0, n_chunks)
        def _(j):
            base = base_w + j * CHUNK
            pltpu.sync_copy(ids_hbm.at[pl.ds(base, CHUNK)], ids_v)
            pltpu.sync_copy(prev_hbm.at[pl.ds(base, CHUNK)], prev_v)

            for t in range(NG):
                for kk in range(GW // L):
                    sl = pl.ds(t * GW + kk * L, L)
                    dsl = pl.ds(kk * L, L)
                    ids16 = ids_v[sl]
                    prev16 = prev_v[sl]
                    uidx_v[t, dsl] = _mod_hs(ids16)
                    bidx_v[t, dsl] = _mod_hs(_mod_hs(prev16) * 31 + ids16)

            cps = []
            for t in range(NG):
                rsl = pl.ds(t * GW, GW)
                cps.append(pltpu.async_copy(
                    uni_hbm.at[uidx_v.at[t]], uni_v.at[rsl], sem_u))
                cps.append(pltpu.async_copy(
                    bi_hbm.at[bidx_v.at[t]], bi_v.at[rsl], sem_b))
            for cp in cps:
                cp.wait()

            pltpu.sync_copy(uni_v, acc_s.at[pl.ds(acc_base, CHUNK)])
            for t in range(NG):
                pltpu.sync_copy(bi_v.at[pl.ds(t * GW, GW)],
                                acc_s.at[lin_v.at[t]], add=True)

            pltpu.sync_copy(acc_s.at[pl.ds(acc_base, CHUNK)],
                            out_hbm.at[pl.ds(base, CHUNK)])

    return k


def kernel(input_ids, unigram_table, bigram_table):
    bt, s = input_ids.shape
    ids = input_ids.astype(jnp.int32)
    prev = jnp.pad(ids[:, :-1], ((0, 0), (1, 0)))
    b = bt * s
    out = _make_kernel(b)(ids.reshape(b), prev.reshape(b),
                          unigram_table, bigram_table)
    return out.reshape(bt, s, D)


# speedup vs baseline: 1.2079x; 1.1752x over previous
import functools

import jax
import jax.numpy as jnp
from jax import lax
from jax.experimental import pallas as pl
from jax.experimental.pallas import tpu as pltpu
from jax.experimental.pallas import tpu_sc as plsc

HS = 300000
D = 64
NC = 2
NS = 16
L = 16
NW = NC * NS
INV_HS = 1.0 / HS
GW = 128
CHUNK = 512
NG = CHUNK // GW


def _mod_hs(x):
    q = (x.astype(jnp.float32) * INV_HS).astype(jnp.int32)
    r = x - q * HS
    r = jnp.where(r >= HS, r - HS, r)
    r = jnp.where(r < 0, r + HS, r)
    return r


@functools.cache
def _make_kernel(B):
    n_per_w = B // NW
    n_chunks = n_per_w // CHUNK
    assert n_chunks % 2 == 0
    mesh = plsc.VectorSubcoreMesh(core_axis_name="c", subcore_axis_name="s")

    @functools.partial(
        pl.kernel,
        out_type=jax.ShapeDtypeStruct((B, D), jnp.float32),
        mesh=mesh,
        compiler_params=pltpu.CompilerParams(use_tc_tiling_on_sc=False),
        scratch_types=[
            pltpu.VMEM((CHUNK,), jnp.int32),
            pltpu.VMEM((CHUNK,), jnp.int32),
            pltpu.VMEM((NG, GW), jnp.int32),
            pltpu.VMEM((NG, GW), jnp.int32),
            pltpu.VMEM((NG, GW), jnp.int32),
            pltpu.VMEM((NG, GW), jnp.int32),
            pltpu.VMEM((CHUNK, D), jnp.float32),
            pltpu.VMEM((CHUNK, D), jnp.float32),
            pltpu.SemaphoreType.DMA,
            pltpu.SemaphoreType.DMA,
            pltpu.SemaphoreType.DMA,
            pltpu.SemaphoreType.DMA,
            pltpu.SemaphoreType.DMA,
            pltpu.SemaphoreType.DMA,
        ],
    )
    def k(ids_hbm, prev_hbm, uni_hbm, bi_hbm, out_hbm,
          ids_v, prev_v, uidx0, bidx0, uidx1, bidx1, buf0, buf1,
          su0, su1, sb0, sb1, sw0, sw1):
        wid = lax.axis_index("s") * NC + lax.axis_index("c")
        base_w = wid * n_per_w
        slots = ((uidx0, bidx0, buf0, su0, sb0, sw0),
                 (uidx1, bidx1, buf1, su1, sb1, sw1))

        def compute_idx(j, uidx, bidx):
            base = base_w + j * CHUNK
            pltpu.sync_copy(ids_hbm.at[pl.ds(base, CHUNK)], ids_v)
            pltpu.sync_copy(prev_hbm.at[pl.ds(base, CHUNK)], prev_v)
            for t in range(NG):
                for kk in range(GW // L):
                    sl = pl.ds(t * GW + kk * L, L)
                    dsl = pl.ds(kk * L, L)
                    ids16 = ids_v[sl]
                    prev16 = prev_v[sl]
                    uidx[t, dsl] = _mod_hs(ids16)
                    bidx[t, dsl] = _mod_hs(_mod_hs(prev16) * 31 + ids16)

        def fire_uni(uidx, buf, sem):
            return [pltpu.async_copy(uni_hbm.at[uidx.at[t]],
                                     buf.at[pl.ds(t * GW, GW)], sem)
                    for t in range(NG)]

        def fire_bi_add(bidx, buf, sem):
            return [pltpu.async_copy(bi_hbm.at[bidx.at[t]],
                                     buf.at[pl.ds(t * GW, GW)], sem, add=True)
                    for t in range(NG)]

        def drain(cps):
            for cp in cps:
                cp.wait()

        def writeout(j, buf, sem):
            base = base_w + j * CHUNK
            return pltpu.async_copy(buf, out_hbm.at[pl.ds(base, CHUNK)], sem)

        def drain_write(buf, sw):
            pltpu.make_async_copy(buf, out_hbm.at[pl.ds(0, CHUNK)], sw).wait()

        uidx, bidx, buf, su, sb, sw = slots[0]
        compute_idx(0, uidx, bidx)
        fire_uni(uidx, buf, su)

        @pl.loop(0, n_chunks, step=2)
        def _(j):
            for p in range(2):
                uidx, bidx, buf, su, sb, sw = slots[p]
                uidx_n, bidx_n, buf_n, su_n, sb_n, sw_n = slots[1 - p]
                jj = j + p

                pltpu.make_async_copy(
                    uni_hbm.at[pl.ds(0, CHUNK)], buf, su).wait()
                cps_b = fire_bi_add(bidx, buf, sb)
                @pl.when(jj + 1 < n_chunks)
                def _():
                    @pl.when(jj >= 1)
                    def _():
                        drain_write(buf_n, sw_n)
                    compute_idx(jj + 1, uidx_n, bidx_n)
                    fire_uni(uidx_n, buf_n, su_n)
                drain(cps_b)
                writeout(jj, buf, sw)

        for p in range(2):
            uidx, bidx, buf, su, sb, sw = slots[p]
            drain_write(buf, sw)

    return k


def kernel(input_ids, unigram_table, bigram_table):
    bt, s = input_ids.shape
    ids = input_ids.astype(jnp.int32)
    prev = jnp.pad(ids[:, :-1], ((0, 0), (1, 0)))
    b = bt * s
    out = _make_kernel(b)(ids.reshape(b), prev.reshape(b),
                          unigram_table, bigram_table)
    return out.reshape(bt, s, D)
